# baseline (speedup 1.0000x reference)
import functools

import jax
import jax.numpy as jnp
from jax import lax
from jax.experimental import pallas as pl
from jax.experimental.pallas import tpu as pltpu
from jax.experimental.pallas import tpu_sc as plsc

VOCAB = 151936
HIDDEN = 1536

NC = 2
NS = 16
NW = NC * NS

B_TOTAL = 4 * 4096
B_PER_W = B_TOTAL // NW
CHUNK = 16
NBUF = 4
LOOK = 3
NCHUNK = B_PER_W // CHUNK


def _gather_body(table_hbm, idx_hbm, out_hbm, idx_v, rows_v, gsem, osem):
  wid = lax.axis_index("s") * NC + lax.axis_index("c")
  base = wid * B_PER_W
  pltpu.sync_copy(idx_hbm.at[pl.ds(base, B_PER_W)], idx_v)

  gathers = [None] * NCHUNK
  outs = [None] * NCHUNK

  def start_out(j):
    gathers[j].wait()
    outs[j] = pltpu.async_copy(
        rows_v.at[j % NBUF], out_hbm.at[pl.ds(base + j * CHUNK, CHUNK)], osem)

  for i in range(NCHUNK):
    if i >= NBUF:
      outs[i - NBUF].wait()
    gathers[i] = pltpu.async_copy(
        table_hbm.at[idx_v.at[pl.ds(i * CHUNK, CHUNK)]],
        rows_v.at[i % NBUF], gsem)
    if i >= LOOK:
      start_out(i - LOOK)
  for j in range(NCHUNK - LOOK, NCHUNK):
    start_out(j)
  for j in range(NCHUNK - NBUF, NCHUNK):
    outs[j].wait()


@jax.jit
def _gather(table, idx):
  mesh = plsc.VectorSubcoreMesh(core_axis_name="c", subcore_axis_name="s")
  f = pl.kernel(
      _gather_body,
      out_type=jax.ShapeDtypeStruct((B_TOTAL, HIDDEN), jnp.float32),
      mesh=mesh,
      scratch_types=[
          pltpu.VMEM((B_PER_W,), jnp.int32),
          pltpu.VMEM((NBUF, CHUNK, HIDDEN), jnp.float32),
          pltpu.SemaphoreType.DMA,
          pltpu.SemaphoreType.DMA,
      ],
  )
  return f(table, idx)


def kernel(input_ids, table):
  ids = input_ids.reshape(-1).astype(jnp.int32)
  out = _gather(table, ids)
  return out.reshape(input_ids.shape + (HIDDEN,))


# speedup vs baseline: 1.0111x; 1.0111x over previous
import functools

import jax
import jax.numpy as jnp
from jax import lax
from jax.experimental import pallas as pl
from jax.experimental.pallas import tpu as pltpu
from jax.experimental.pallas import tpu_sc as plsc

VOCAB = 151936
HIDDEN = 1536

NC = 2
NS = 16
NW = NC * NS

B_TOTAL = 4 * 4096
B_PER_W = B_TOTAL // NW
CHUNK = 16
NBUF = 5
LOOK = 1
NCHUNK = B_PER_W // CHUNK


def _gather_body(table_hbm, idx_hbm, out_hbm, idx_v, rows_v, gsem, osem):
  wid = lax.axis_index("s") * NC + lax.axis_index("c")
  base = wid * B_PER_W
  pltpu.sync_copy(idx_hbm.at[pl.ds(base, B_PER_W)], idx_v)

  gathers = [None] * NCHUNK
  outs = [None] * NCHUNK

  def start_out(j):
    gathers[j].wait()
    outs[j] = pltpu.async_copy(
        rows_v.at[j % NBUF], out_hbm.at[pl.ds(base + j * CHUNK, CHUNK)], osem)

  for i in range(NCHUNK):
    if i >= NBUF:
      outs[i - NBUF].wait()
    gathers[i] = pltpu.async_copy(
        table_hbm.at[idx_v.at[pl.ds(i * CHUNK, CHUNK)]],
        rows_v.at[i % NBUF], gsem)
    if i >= LOOK:
      start_out(i - LOOK)
  for j in range(NCHUNK - LOOK, NCHUNK):
    start_out(j)
  for j in range(NCHUNK - NBUF, NCHUNK):
    outs[j].wait()


@jax.jit
def _gather(table, idx):
  mesh = plsc.VectorSubcoreMesh(core_axis_name="c", subcore_axis_name="s")
  f = pl.kernel(
      _gather_body,
      out_type=jax.ShapeDtypeStruct((B_TOTAL, HIDDEN), jnp.float32),
      mesh=mesh,
      scratch_types=[
          pltpu.VMEM((B_PER_W,), jnp.int32),
          pltpu.VMEM((NBUF, CHUNK, HIDDEN), jnp.float32),
          pltpu.SemaphoreType.DMA,
          pltpu.SemaphoreType.DMA,
      ],
  )
  return f(table, idx)


def kernel(input_ids, table):
  ids = input_ids.reshape(-1).astype(jnp.int32)
  out = _gather(table, ids)
  return out.reshape(input_ids.shape + (HIDDEN,))
